# baseline (speedup 1.0000x reference)
import jax
import jax.numpy as jnp
from jax.experimental import pallas as pl
from jax.experimental.pallas import tpu as pltpu

_B = 64
_NN = 64
_FEAT = 128
_OUT = 64
_HEADS = 4
_CAPS = 10
_K = 8


def _lrelu(x):
    return jnp.maximum(x, 0.2 * x)


def _tree_sum0(x):
    n = x.shape[0]
    while n > 1:
        h = n // 2
        x = x[:h] + x[h:n]
        n = h
    return x[0]


def _tree_max0(x):
    n = x.shape[0]
    while n > 1:
        h = n // 2
        x = jnp.maximum(x[:h], x[h:n])
        n = h
    return x[0]


def _layer_norm(x, g, b):
    mu = jnp.mean(x, axis=-1, keepdims=True)
    var = jnp.mean((x - mu) ** 2, axis=-1, keepdims=True)
    return (x - mu) / jnp.sqrt(var + 1e-5) * g + b


def _gat_dense(x, adj3, Wl, bl, Wr, br, S, bias):
    f = S.shape[0]
    xl = jnp.dot(x, Wl, preferred_element_type=jnp.float32) + bl
    xr = jnp.dot(x, Wr, preferred_element_type=jnp.float32) + br
    xl16 = xl.astype(jnp.bfloat16)
    xr16 = xr.astype(jnp.bfloat16)
    er = _lrelu(xl16[:, None, :] + xr16[None, :, :])
    lg = jnp.dot(er.reshape(_NN * _NN, f), S.astype(jnp.bfloat16),
                 preferred_element_type=jnp.float32).reshape(_NN, _NN, f)
    mx = _tree_max0(lg)[None]
    ex = jnp.exp(lg - mx) * adj3[:, :, :f]
    den = _tree_sum0(ex)
    num = _tree_sum0(ex * xl[:, None, :])
    return num / (den + 1e-16) + bias


def _gnn_body(fin_ref, adj_ref, wemb_ref, bemb_ref,
              wl1_ref, bl1_ref, wr1_ref, br1_ref, s1_ref, bias1_ref,
              g1_ref, be1_ref,
              wl2_ref, bl2_ref, wr2_ref, br2_ref, s2_ref, bias2_ref,
              g2_ref, be2_ref,
              wl3_ref, bl3_ref, wr3_ref, br3_ref, s3_ref, bias3_ref,
              out_ref):
    for s in range(_K):
        fin = fin_ref[s]
        adj3 = adj_ref[s][:, :, None] * jnp.ones((1, 1, _FEAT), jnp.float32)
        x = jnp.dot(fin, wemb_ref[...], preferred_element_type=jnp.float32) + bemb_ref[...]
        h = _gat_dense(x, adj3, wl1_ref[...], bl1_ref[...], wr1_ref[...], br1_ref[...],
                       s1_ref[...], bias1_ref[...])
        h = jax.nn.relu(_layer_norm(h, g1_ref[...], be1_ref[...]))
        h = _gat_dense(h, adj3, wl2_ref[...], bl2_ref[...], wr2_ref[...], br2_ref[...],
                       s2_ref[...], bias2_ref[...])
        h = jax.nn.relu(_layer_norm(h, g2_ref[...], be2_ref[...]))
        o = _gat_dense(h, adj3, wl3_ref[...], bl3_ref[...], wr3_ref[...], br3_ref[...],
                       s3_ref[...], bias3_ref[...])
        out_ref[s] = o


def kernel(obs, aux, team_obs, target_obs, team_mask, local_target_mask,
           W_emb, b_emb, Wl1, bl1, Wr1, br1, att1, bias1, g1, be1,
           Wl2, bl2, Wr2, br2, att2, bias2, g2, be2,
           Wl3, bl3, Wr3, br3, att3, bias3):
    batch, n_agents, _ = team_obs.shape
    team = jnp.concatenate(
        [team_obs, jnp.zeros((batch, n_agents, 2), dtype=team_obs.dtype)], axis=2)
    t = target_obs[:, :, :-1]
    t2 = jnp.concatenate(
        [t[:, :, :2 + _CAPS], jnp.zeros((batch, t.shape[1], 2), dtype=t.dtype),
         t[:, :, -2:]], axis=2)
    fin_in = jnp.concatenate([team, t2], axis=1)
    imp_team = team_mask[:, 0] != -1000000000.0
    loc = local_target_mask[:, :-1] != 0
    valid = jnp.concatenate([imp_team, loc], axis=1)
    eye = jnp.eye(_NN, dtype=bool)
    adj = (valid[:, :, None] & valid[:, None, :] & ~eye[None]) | eye[None]
    adjf = adj.astype(jnp.float32)

    c = _FEAT // _HEADS
    hid = jnp.arange(_FEAT) // c
    blk = (hid[:, None] == hid[None, :]).astype(jnp.float32)
    S1 = att1.reshape(_FEAT, 1) * blk
    S2 = att2.reshape(_FEAT, 1) * blk
    S3 = att3.reshape(_OUT, 1) * jnp.ones((_OUT, _OUT), jnp.float32)

    row = lambda v: v.reshape(1, -1)
    grid = (batch // _K,)
    bmap = lambda b: (b, 0, 0)
    wmap2 = lambda b: (0, 0)
    full2 = lambda a: pl.BlockSpec(a.shape, wmap2)

    args = [
        (fin_in, pl.BlockSpec((_K, _NN, 16), bmap)),
        (adjf, pl.BlockSpec((_K, _NN, _NN), bmap)),
        (W_emb, full2(W_emb)),
        (row(b_emb), full2(row(b_emb))),
        (Wl1, full2(Wl1)), (row(bl1), full2(row(bl1))),
        (Wr1, full2(Wr1)), (row(br1), full2(row(br1))),
        (S1, full2(S1)), (row(bias1), full2(row(bias1))),
        (row(g1), full2(row(g1))), (row(be1), full2(row(be1))),
        (Wl2, full2(Wl2)), (row(bl2), full2(row(bl2))),
        (Wr2, full2(Wr2)), (row(br2), full2(row(br2))),
        (S2, full2(S2)), (row(bias2), full2(row(bias2))),
        (row(g2), full2(row(g2))), (row(be2), full2(row(be2))),
        (Wl3, full2(Wl3)), (row(bl3), full2(row(bl3))),
        (Wr3, full2(Wr3)), (row(br3), full2(row(br3))),
        (S3, full2(S3)), (row(bias3), full2(row(bias3))),
    ]
    operands = [a for a, _ in args]
    in_specs = [s for _, s in args]

    out = pl.pallas_call(
        _gnn_body,
        grid=grid,
        compiler_params=pltpu.CompilerParams(
            dimension_semantics=("parallel",)),
        in_specs=in_specs,
        out_specs=pl.BlockSpec((_K, _NN, _OUT), bmap),
        out_shape=jax.ShapeDtypeStruct((batch, _NN, _OUT), jnp.float32),
    )(*operands)
    return out


# speedup vs baseline: 1.0273x; 1.0273x over previous
import jax
import jax.numpy as jnp
from jax.experimental import pallas as pl
from jax.experimental.pallas import tpu as pltpu

_B = 64
_NN = 64
_FEAT = 128
_OUT = 64
_HEADS = 4
_CAPS = 10
_K = 8


def _lrelu(x):
    return jnp.maximum(x, 0.2 * x)


def _tree_sum0(x):
    n = x.shape[0]
    while n > 1:
        h = n // 2
        x = x[:h] + x[h:n]
        n = h
    return x[0]


def _tree_max0(x):
    n = x.shape[0]
    while n > 1:
        h = n // 2
        x = jnp.maximum(x[:h], x[h:n])
        n = h
    return x[0]


def _layer_norm(x, g, b):
    mu = jnp.mean(x, axis=-1, keepdims=True)
    var = jnp.mean((x - mu) ** 2, axis=-1, keepdims=True)
    return (x - mu) / jnp.sqrt(var + 1e-5) * g + b


def _att_core(xl, xr, adjm, S, bias):
    f = S.shape[0]
    xl16 = xl.astype(jnp.bfloat16)
    xr16 = xr.astype(jnp.bfloat16)
    er = _lrelu(xl16[:, None, :] + xr16[None, :, :])
    lg = jnp.dot(er.reshape(_NN * _NN, f), S.astype(jnp.bfloat16),
                 preferred_element_type=jnp.float32).reshape(_NN, _NN, f)
    mx = _tree_max0(lg)[None]
    ex = jnp.exp(lg - mx) * adjm
    den = _tree_sum0(ex)
    num = _tree_sum0(ex * xl[:, None, :])
    return num / (den + 1e-16) + bias


def _gat_dense(x, adj3, Wl, bl, Wr, br, S, bias):
    xl = jnp.dot(x, Wl, preferred_element_type=jnp.float32) + bl
    xr = jnp.dot(x, Wr, preferred_element_type=jnp.float32) + br
    return _att_core(xl, xr, adj3[:, :, :S.shape[0]], S, bias)


def _gnn_body(fin_ref, adj_ref, wemb_ref, bemb_ref,
              wl1_ref, bl1_ref, wr1_ref, br1_ref, s1_ref, bias1_ref,
              g1_ref, be1_ref,
              wl2_ref, bl2_ref, wr2_ref, br2_ref, s2_ref, bias2_ref,
              g2_ref, be2_ref,
              wl3_ref, bl3_ref, wr3_ref, br3_ref, s3_ref, bias3_ref,
              out_ref):
    hs = []
    adjs = []
    for s in range(_K):
        fin = fin_ref[s]
        adj3 = adj_ref[s][:, :, None] * jnp.ones((1, 1, _FEAT), jnp.float32)
        x = jnp.dot(fin, wemb_ref[...], preferred_element_type=jnp.float32) + bemb_ref[...]
        h = _gat_dense(x, adj3, wl1_ref[...], bl1_ref[...], wr1_ref[...], br1_ref[...],
                       s1_ref[...], bias1_ref[...])
        h = jax.nn.relu(_layer_norm(h, g1_ref[...], be1_ref[...]))
        h = _gat_dense(h, adj3, wl2_ref[...], bl2_ref[...], wr2_ref[...], br2_ref[...],
                       s2_ref[...], bias2_ref[...])
        h = jax.nn.relu(_layer_norm(h, g2_ref[...], be2_ref[...]))
        hs.append(h)
        adjs.append(adj3)
    for s in range(0, _K, 2):
        xl_p = jnp.concatenate(
            [jnp.dot(hs[s], wl3_ref[...], preferred_element_type=jnp.float32),
             jnp.dot(hs[s + 1], wl3_ref[...], preferred_element_type=jnp.float32)],
            axis=-1) + bl3_ref[...]
        xr_p = jnp.concatenate(
            [jnp.dot(hs[s], wr3_ref[...], preferred_element_type=jnp.float32),
             jnp.dot(hs[s + 1], wr3_ref[...], preferred_element_type=jnp.float32)],
            axis=-1) + br3_ref[...]
        adj_p = jnp.concatenate(
            [adjs[s][:, :, :_OUT], adjs[s + 1][:, :, :_OUT]], axis=-1)
        o = _att_core(xl_p, xr_p, adj_p, s3_ref[...], bias3_ref[...])
        out_ref[s] = o[:, :_OUT]
        out_ref[s + 1] = o[:, _OUT:]


def kernel(obs, aux, team_obs, target_obs, team_mask, local_target_mask,
           W_emb, b_emb, Wl1, bl1, Wr1, br1, att1, bias1, g1, be1,
           Wl2, bl2, Wr2, br2, att2, bias2, g2, be2,
           Wl3, bl3, Wr3, br3, att3, bias3):
    batch, n_agents, _ = team_obs.shape
    team = jnp.concatenate(
        [team_obs, jnp.zeros((batch, n_agents, 2), dtype=team_obs.dtype)], axis=2)
    t = target_obs[:, :, :-1]
    t2 = jnp.concatenate(
        [t[:, :, :2 + _CAPS], jnp.zeros((batch, t.shape[1], 2), dtype=t.dtype),
         t[:, :, -2:]], axis=2)
    fin_in = jnp.concatenate([team, t2], axis=1)
    imp_team = team_mask[:, 0] != -1000000000.0
    loc = local_target_mask[:, :-1] != 0
    valid = jnp.concatenate([imp_team, loc], axis=1)
    eye = jnp.eye(_NN, dtype=bool)
    adj = (valid[:, :, None] & valid[:, None, :] & ~eye[None]) | eye[None]
    adjf = adj.astype(jnp.float32)

    c = _FEAT // _HEADS
    hid = jnp.arange(_FEAT) // c
    blk = (hid[:, None] == hid[None, :]).astype(jnp.float32)
    S1 = att1.reshape(_FEAT, 1) * blk
    S2 = att2.reshape(_FEAT, 1) * blk
    a3 = att3.reshape(_OUT, 1) * jnp.ones((_OUT, _OUT), jnp.float32)
    z3 = jnp.zeros((_OUT, _OUT), jnp.float32)
    S3 = jnp.concatenate(
        [jnp.concatenate([a3, z3], axis=1), jnp.concatenate([z3, a3], axis=1)],
        axis=0)
    bl3 = jnp.concatenate([bl3, bl3])
    br3 = jnp.concatenate([br3, br3])
    bias3 = jnp.concatenate([bias3, bias3])

    row = lambda v: v.reshape(1, -1)
    grid = (batch // _K,)
    bmap = lambda b: (b, 0, 0)
    wmap2 = lambda b: (0, 0)
    full2 = lambda a: pl.BlockSpec(a.shape, wmap2)

    args = [
        (fin_in, pl.BlockSpec((_K, _NN, 16), bmap)),
        (adjf, pl.BlockSpec((_K, _NN, _NN), bmap)),
        (W_emb, full2(W_emb)),
        (row(b_emb), full2(row(b_emb))),
        (Wl1, full2(Wl1)), (row(bl1), full2(row(bl1))),
        (Wr1, full2(Wr1)), (row(br1), full2(row(br1))),
        (S1, full2(S1)), (row(bias1), full2(row(bias1))),
        (row(g1), full2(row(g1))), (row(be1), full2(row(be1))),
        (Wl2, full2(Wl2)), (row(bl2), full2(row(bl2))),
        (Wr2, full2(Wr2)), (row(br2), full2(row(br2))),
        (S2, full2(S2)), (row(bias2), full2(row(bias2))),
        (row(g2), full2(row(g2))), (row(be2), full2(row(be2))),
        (Wl3, full2(Wl3)), (row(bl3), full2(row(bl3))),
        (Wr3, full2(Wr3)), (row(br3), full2(row(br3))),
        (S3, full2(S3)), (row(bias3), full2(row(bias3))),
    ]
    operands = [a for a, _ in args]
    in_specs = [s for _, s in args]

    out = pl.pallas_call(
        _gnn_body,
        grid=grid,
        compiler_params=pltpu.CompilerParams(
            dimension_semantics=("parallel",)),
        in_specs=in_specs,
        out_specs=pl.BlockSpec((_K, _NN, _OUT), bmap),
        out_shape=jax.ShapeDtypeStruct((batch, _NN, _OUT), jnp.float32),
    )(*operands)
    return out
